# baseline (speedup 1.0000x reference)
import jax
import jax.numpy as jnp
from jax import lax
from jax.experimental import pallas as pl
from jax.experimental.pallas import tpu as pltpu
from jax.experimental.pallas import tpu_sc as plsc

N = 10000
E = 320000
NFEAT = 128
NHID = 128
NCLASS = 64
HF = NFEAT // 2

NC = 2
NS = 16
NW = NC * NS
CH = 80
NCHA = E // NS // CH
NCHB = E // NW // CH
NPAD = 10240
RPT = NPAD // NS
ZR = 64

_HIGH = lax.Precision.HIGHEST
_SC_PARAMS = pltpu.CompilerParams(use_tc_tiling_on_sc=False)


def _memset_rows(ref, nrows, ncols, value):
  def row(r, _):
    def col(c, _):
      ref[r, pl.ds(c * 16, 16)] = jnp.full((16,), value, jnp.float32)
      return 0
    return lax.fori_loop(0, ncols // 16, col, 0)
  lax.fori_loop(0, nrows, row, 0)


def _agg_pipeline(nch, slots, lead, feat, src_v, dst_v, rows, acc,
                  gsem, ssem, deg_step=None):
  def gather(j, slot):
    pltpu.async_copy(feat.at[src_v.at[j]], rows.at[slot], gsem)

  def wait_gather(slot):
    pltpu.make_async_copy(feat.at[src_v.at[0]], rows.at[slot], gsem).wait()

  def scatter(j, slot):
    pltpu.async_copy(rows.at[slot], acc.at[dst_v.at[j]], ssem, add=True)

  def drain_scatter(slot):
    pltpu.make_async_copy(rows.at[slot], acc.at[dst_v.at[0]], ssem).wait()

  for b in range(lead):
    gather(b, b)

  lag = slots - lead

  def round_(o, _):
    for b in range(slots):
      k = o * slots + b
      wait_gather(b)
      scatter(k, b)
      if deg_step is not None:
        deg_step(k)

      @pl.when(k >= lag)
      def _():
        drain_scatter(b)

      @pl.when(k + lead < nch)
      def _():
        gather(k + lead, (b + lead) % slots)
    return 0
  lax.fori_loop(0, nch // slots, round_, 0)
  for _ in range(lag):
    drain_scatter(0)


def _make_sc_layer1():
  mesh = plsc.VectorSubcoreMesh(core_axis_name="c", subcore_axis_name="s",
                                num_cores=NC, num_subcores=NS)
  out_type = [
      jax.ShapeDtypeStruct((NPAD, NFEAT), jnp.float32),
      jax.ShapeDtypeStruct((NC * NPAD, 16), jnp.float32),
  ]
  slots, lead = 5, 3
  scratch = [
      pltpu.VMEM((NCHA, CH), jnp.int32),
      pltpu.VMEM((NCHA, CH), jnp.int32),
      pltpu.VMEM((slots, CH, HF), jnp.float32),
      pltpu.VMEM((ZR, HF), jnp.float32),
      pltpu.VMEM((CH, 16), jnp.float32),
      pltpu.VMEM((ZR, 16), jnp.float32),
      pltpu.VMEM_SHARED((NPAD, HF), jnp.float32),
      pltpu.VMEM_SHARED((NPAD, 16), jnp.float32),
      pltpu.SemaphoreType.DMA,
      pltpu.SemaphoreType.DMA,
      pltpu.SemaphoreType.DMA,
  ]

  def body(feat2, srcw2, dstw, agg_out, deg_out,
           src_v, dst_v, rows, zbuf, ones_v, zd, acc, dacc,
           gsem, ssem, dsem):
    cid = lax.axis_index("c")
    sid = lax.axis_index("s")

    _memset_rows(zbuf, ZR, HF, 0.0)
    _memset_rows(ones_v, CH, 16, 1.0)
    _memset_rows(zd, ZR, 16, 0.0)

    base = sid * RPT
    for k in range(RPT // ZR):
      pltpu.sync_copy(zbuf, acc.at[pl.ds(base + k * ZR, ZR)])
      pltpu.sync_copy(zd, dacc.at[pl.ds(base + k * ZR, ZR)])
    plsc.subcore_barrier()

    pltpu.sync_copy(srcw2.at[cid * NS + sid], src_v)
    pltpu.sync_copy(dstw.at[sid], dst_v)

    def deg_step(k):
      @pl.when(k % 2 == cid)
      def _():
        pltpu.async_copy(ones_v, dacc.at[dst_v.at[k]], dsem, add=True)

      @pl.when((k % 2 == cid) & (k >= 2))
      def _():
        pltpu.make_async_copy(ones_v, dacc.at[dst_v.at[0]], dsem).wait()

    _agg_pipeline(NCHA, slots, lead, feat2, src_v, dst_v, rows, acc,
                  gsem, ssem, deg_step)
    pltpu.make_async_copy(ones_v, dacc.at[dst_v.at[0]], dsem).wait()
    plsc.subcore_barrier()

    pltpu.sync_copy(acc.at[pl.ds(base, RPT)],
                    agg_out.at[pl.ds(base, RPT), pl.ds(cid * HF, HF)])
    obase = cid * NPAD + base
    pltpu.sync_copy(dacc.at[pl.ds(base, RPT)], deg_out.at[pl.ds(obase, RPT)])

  return pl.kernel(body, out_type=out_type, mesh=mesh, scratch_types=scratch,
                   compiler_params=_SC_PARAMS)


def _make_sc_layer2():
  mesh = plsc.VectorSubcoreMesh(core_axis_name="c", subcore_axis_name="s",
                                num_cores=NC, num_subcores=NS)
  out_type = [jax.ShapeDtypeStruct((NPAD, 2 * NCLASS), jnp.float32)]
  slots, lead = 5, 3
  scratch = [
      pltpu.VMEM((NCHB, CH), jnp.int32),
      pltpu.VMEM((NCHB, CH), jnp.int32),
      pltpu.VMEM((slots, CH, NCLASS), jnp.float32),
      pltpu.VMEM((ZR, NCLASS), jnp.float32),
      pltpu.VMEM_SHARED((NPAD, NCLASS), jnp.float32),
      pltpu.SemaphoreType.DMA,
      pltpu.SemaphoreType.DMA,
  ]

  def body(feat, srcw, dstw, agg_out, src_v, dst_v, rows, zbuf, acc,
           gsem, ssem):
    cid = lax.axis_index("c")
    sid = lax.axis_index("s")
    wid = sid * NC + cid

    _memset_rows(zbuf, ZR, NCLASS, 0.0)
    base = sid * RPT
    for k in range(RPT // ZR):
      pltpu.sync_copy(zbuf, acc.at[pl.ds(base + k * ZR, ZR)])
    plsc.subcore_barrier()

    pltpu.sync_copy(srcw.at[wid], src_v)
    pltpu.sync_copy(dstw.at[wid], dst_v)

    _agg_pipeline(NCHB, slots, lead, feat, src_v, dst_v, rows, acc,
                  gsem, ssem)
    plsc.subcore_barrier()

    pltpu.sync_copy(acc.at[pl.ds(base, RPT)],
                    agg_out.at[pl.ds(base, RPT), pl.ds(cid * NCLASS, NCLASS)])

  return pl.kernel(body, out_type=out_type, mesh=mesh, scratch_types=scratch,
                   compiler_params=_SC_PARAMS)


_sc_cache = {}


def _sc_layer1():
  if "a" not in _sc_cache:
    _sc_cache["a"] = _make_sc_layer1()
  return _sc_cache["a"]


def _sc_layer2():
  if "b" not in _sc_cache:
    _sc_cache["b"] = _make_sc_layer2()
  return _sc_cache["b"]


BN = 2000


def _deg_inv(degp_ref):
  deg = degp_ref[0, :, 0:1] + degp_ref[1, :, 0:1]
  return 1.0 / jnp.maximum(deg, 1.0)


def _tc_mid_body(x_r, aggp_r, degp_r, w1s_r, w1n_r, b1_r, w2s_r, w2n_r,
                 b2_r, y2_r, self2_r):
  inv = _deg_inv(degp_r)
  agg = aggp_r[...] * inv
  x = x_r[...]
  h = (jnp.dot(x, w1s_r[...], preferred_element_type=jnp.float32,
               precision=_HIGH)
       + jnp.dot(agg, w1n_r[...], preferred_element_type=jnp.float32,
                 precision=_HIGH)
       + b1_r[...][None, :])
  h = jnp.maximum(h, 0.0)
  y2 = jnp.dot(h, w2n_r[...], preferred_element_type=jnp.float32,
               precision=_HIGH)
  y2_r[...] = jnp.concatenate(
      [y2, jnp.zeros((BN, NCLASS), jnp.float32)], axis=1)
  self2_r[...] = (jnp.dot(h, w2s_r[...], preferred_element_type=jnp.float32,
                          precision=_HIGH) + b2_r[...][None, :])


def _row_block(d):
  return pl.BlockSpec((BN, d), lambda i: (i, 0))


def _split_block(d):
  return pl.BlockSpec((2, BN, d), lambda i: (0, i, 0))


def _full_block(shape):
  nd = len(shape)
  return pl.BlockSpec(shape, (lambda i: (0,) * nd))


_tc_mid = pl.pallas_call(
    _tc_mid_body,
    grid=(N // BN,),
    in_specs=[
        _row_block(NFEAT),
        _row_block(NFEAT),
        _split_block(16),
        _full_block((NFEAT, NHID)),
        _full_block((NFEAT, NHID)),
        _full_block((NHID,)),
        _full_block((NHID, NCLASS)),
        _full_block((NHID, NCLASS)),
        _full_block((NCLASS,)),
    ],
    out_specs=[_row_block(2 * NCLASS), _row_block(NCLASS)],
    out_shape=[
        jax.ShapeDtypeStruct((N, 2 * NCLASS), jnp.float32),
        jax.ShapeDtypeStruct((N, NCLASS), jnp.float32),
    ],
)


def _tc_out_body(self2_r, aggp2_r, degp_r, out_r):
  inv = _deg_inv(degp_r)
  logits = self2_r[...] + (
      aggp2_r[:, 0:NCLASS] + aggp2_r[:, NCLASS:2 * NCLASS]) * inv
  m = jnp.max(logits, axis=1, keepdims=True)
  ex = jnp.exp(logits - m)
  lse = jnp.log(jnp.sum(ex, axis=1, keepdims=True)) + m
  out_r[...] = logits - lse


_tc_out = pl.pallas_call(
    _tc_out_body,
    grid=(N // BN,),
    in_specs=[_row_block(NCLASS), _row_block(2 * NCLASS), _split_block(16)],
    out_specs=_row_block(NCLASS),
    out_shape=jax.ShapeDtypeStruct((N, NCLASS), jnp.float32),
)


def kernel(x, edge_index, W1_self, W1_neigh, b1, W2_self, W2_neigh, b2):
  e = edge_index.astype(jnp.int32).reshape(2 * E)
  src, dst = e[0:E], e[E:2 * E]
  feat2 = x.reshape(2 * N, HF)
  src2 = (src * 2).reshape(NS, NCHA, CH)
  srcw2 = jnp.stack([src2, src2 + 1]).reshape(NC * NS, NCHA, CH)
  dstw_a = dst.reshape(NS, NCHA, CH)
  aggx, degp = _sc_layer1()(feat2, srcw2, dstw_a)
  degp = degp.reshape(NC, NPAD, 16)
  y2, self2 = _tc_mid(x, aggx, degp, W1_self, W1_neigh, b1,
                      W2_self, W2_neigh, b2)
  y2v = y2.reshape(2 * N, NCLASS)
  srcw_b = (src * 2).reshape(NW, NCHB, CH)
  dstw_b = dst.reshape(NW, NCHB, CH)
  (agg2,) = _sc_layer2()(y2v, srcw_b, dstw_b)
  return _tc_out(self2, agg2, degp)


# speedup vs baseline: 1.1395x; 1.1395x over previous
import jax
import jax.numpy as jnp
from jax import lax
from jax.experimental import pallas as pl
from jax.experimental.pallas import tpu as pltpu
from jax.experimental.pallas import tpu_sc as plsc

N = 10000
E = 320000
NFEAT = 128
NHID = 128
NCLASS = 64
HF = NFEAT // 2

NC = 2
NS = 16
NW = NC * NS
CH = 80
NCHA = E // NS // CH
NCHB = E // NW // CH
NPAD = 10240
RPT = NPAD // NS
ZR = 64

_HIGH = lax.Precision.DEFAULT
_SC_PARAMS = pltpu.CompilerParams(use_tc_tiling_on_sc=False)


def _memset_rows(ref, nrows, ncols, value):
  def row(r, _):
    def col(c, _):
      ref[r, pl.ds(c * 16, 16)] = jnp.full((16,), value, jnp.float32)
      return 0
    return lax.fori_loop(0, ncols // 16, col, 0)
  lax.fori_loop(0, nrows, row, 0)


def _agg_pipeline(nch, slots, lead, feat, src_v, dst_v, rows, acc,
                  gsem, ssem, deg_step=None):
  def gather(j, slot):
    pltpu.async_copy(feat.at[src_v.at[j]], rows.at[slot], gsem)

  def wait_gather(slot):
    pltpu.make_async_copy(feat.at[src_v.at[0]], rows.at[slot], gsem).wait()

  def scatter(j, slot):
    pltpu.async_copy(rows.at[slot], acc.at[dst_v.at[j]], ssem, add=True)

  def drain_scatter(slot):
    pltpu.make_async_copy(rows.at[slot], acc.at[dst_v.at[0]], ssem).wait()

  for b in range(lead):
    gather(b, b)

  lag = slots - lead

  def round_(o, _):
    for b in range(slots):
      k = o * slots + b
      wait_gather(b)
      scatter(k, b)
      if deg_step is not None:
        deg_step(k)

      @pl.when(k >= lag)
      def _():
        drain_scatter(b)

      @pl.when(k + lead < nch)
      def _():
        gather(k + lead, (b + lead) % slots)
    return 0
  lax.fori_loop(0, nch // slots, round_, 0)
  for _ in range(lag):
    drain_scatter(0)


def _make_sc_layer1():
  mesh = plsc.VectorSubcoreMesh(core_axis_name="c", subcore_axis_name="s",
                                num_cores=NC, num_subcores=NS)
  out_type = [
      jax.ShapeDtypeStruct((NPAD, NFEAT), jnp.float32),
      jax.ShapeDtypeStruct((NC * NPAD, 16), jnp.float32),
  ]
  slots, lead = 5, 3
  scratch = [
      pltpu.VMEM((NCHA, CH), jnp.int32),
      pltpu.VMEM((NCHA, CH), jnp.int32),
      pltpu.VMEM((slots, CH, HF), jnp.float32),
      pltpu.VMEM((ZR, HF), jnp.float32),
      pltpu.VMEM((CH, 16), jnp.float32),
      pltpu.VMEM((ZR, 16), jnp.float32),
      pltpu.VMEM_SHARED((NPAD, HF), jnp.float32),
      pltpu.VMEM_SHARED((NPAD, 16), jnp.float32),
      pltpu.SemaphoreType.DMA,
      pltpu.SemaphoreType.DMA,
      pltpu.SemaphoreType.DMA,
  ]

  def body(feat2, srcw2, dstw, agg_out, deg_out,
           src_v, dst_v, rows, zbuf, ones_v, zd, acc, dacc,
           gsem, ssem, dsem):
    cid = lax.axis_index("c")
    sid = lax.axis_index("s")

    _memset_rows(zbuf, ZR, HF, 0.0)
    _memset_rows(ones_v, CH, 16, 1.0)
    _memset_rows(zd, ZR, 16, 0.0)

    base = sid * RPT
    for k in range(RPT // ZR):
      pltpu.sync_copy(zbuf, acc.at[pl.ds(base + k * ZR, ZR)])
      pltpu.sync_copy(zd, dacc.at[pl.ds(base + k * ZR, ZR)])
    plsc.subcore_barrier()

    pltpu.sync_copy(srcw2.at[cid * NS + sid], src_v)
    pltpu.sync_copy(dstw.at[sid], dst_v)

    def deg_step(k):
      @pl.when(k % 2 == cid)
      def _():
        pltpu.async_copy(ones_v, dacc.at[dst_v.at[k]], dsem, add=True)

      @pl.when((k % 2 == cid) & (k >= 2))
      def _():
        pltpu.make_async_copy(ones_v, dacc.at[dst_v.at[0]], dsem).wait()

    _agg_pipeline(NCHA, slots, lead, feat2, src_v, dst_v, rows, acc,
                  gsem, ssem, deg_step)
    pltpu.make_async_copy(ones_v, dacc.at[dst_v.at[0]], dsem).wait()
    plsc.subcore_barrier()

    pltpu.sync_copy(acc.at[pl.ds(base, RPT)],
                    agg_out.at[pl.ds(base, RPT), pl.ds(cid * HF, HF)])
    obase = cid * NPAD + base
    pltpu.sync_copy(dacc.at[pl.ds(base, RPT)], deg_out.at[pl.ds(obase, RPT)])

  return pl.kernel(body, out_type=out_type, mesh=mesh, scratch_types=scratch,
                   compiler_params=_SC_PARAMS)


def _make_sc_layer2():
  mesh = plsc.VectorSubcoreMesh(core_axis_name="c", subcore_axis_name="s",
                                num_cores=NC, num_subcores=NS)
  out_type = [jax.ShapeDtypeStruct((NPAD, 2 * NCLASS), jnp.float32)]
  slots, lead = 5, 3
  scratch = [
      pltpu.VMEM((NCHB, CH), jnp.int32),
      pltpu.VMEM((NCHB, CH), jnp.int32),
      pltpu.VMEM((slots, CH, NCLASS), jnp.float32),
      pltpu.VMEM((ZR, NCLASS), jnp.float32),
      pltpu.VMEM_SHARED((NPAD, NCLASS), jnp.float32),
      pltpu.SemaphoreType.DMA,
      pltpu.SemaphoreType.DMA,
  ]

  def body(feat, srcw, dstw, agg_out, src_v, dst_v, rows, zbuf, acc,
           gsem, ssem):
    cid = lax.axis_index("c")
    sid = lax.axis_index("s")
    wid = sid * NC + cid

    _memset_rows(zbuf, ZR, NCLASS, 0.0)
    base = sid * RPT
    for k in range(RPT // ZR):
      pltpu.sync_copy(zbuf, acc.at[pl.ds(base + k * ZR, ZR)])
    plsc.subcore_barrier()

    pltpu.sync_copy(srcw.at[wid], src_v)
    pltpu.sync_copy(dstw.at[wid], dst_v)

    _agg_pipeline(NCHB, slots, lead, feat, src_v, dst_v, rows, acc,
                  gsem, ssem)
    plsc.subcore_barrier()

    pltpu.sync_copy(acc.at[pl.ds(base, RPT)],
                    agg_out.at[pl.ds(base, RPT), pl.ds(cid * NCLASS, NCLASS)])

  return pl.kernel(body, out_type=out_type, mesh=mesh, scratch_types=scratch,
                   compiler_params=_SC_PARAMS)


_sc_cache = {}


def _sc_layer1():
  if "a" not in _sc_cache:
    _sc_cache["a"] = _make_sc_layer1()
  return _sc_cache["a"]


def _sc_layer2():
  if "b" not in _sc_cache:
    _sc_cache["b"] = _make_sc_layer2()
  return _sc_cache["b"]


BN = 2000


def _deg_inv(degp_ref):
  deg = degp_ref[0, :, 0:1] + degp_ref[1, :, 0:1]
  return 1.0 / jnp.maximum(deg, 1.0)


def _tc_mid_body(x_r, aggp_r, degp_r, w1s_r, w1n_r, b1_r, w2s_r, w2n_r,
                 b2_r, y2_r, self2_r):
  inv = _deg_inv(degp_r)
  agg = aggp_r[...] * inv
  x = x_r[...]
  h = (jnp.dot(x, w1s_r[...], preferred_element_type=jnp.float32,
               precision=_HIGH)
       + jnp.dot(agg, w1n_r[...], preferred_element_type=jnp.float32,
                 precision=_HIGH)
       + b1_r[...][None, :])
  h = jnp.maximum(h, 0.0)
  y2 = jnp.dot(h, w2n_r[...], preferred_element_type=jnp.float32,
               precision=_HIGH)
  y2_r[...] = jnp.concatenate(
      [y2, jnp.zeros((BN, NCLASS), jnp.float32)], axis=1)
  self2_r[...] = (jnp.dot(h, w2s_r[...], preferred_element_type=jnp.float32,
                          precision=_HIGH) + b2_r[...][None, :])


def _row_block(d):
  return pl.BlockSpec((BN, d), lambda i: (i, 0))


def _split_block(d):
  return pl.BlockSpec((2, BN, d), lambda i: (0, i, 0))


def _full_block(shape):
  nd = len(shape)
  return pl.BlockSpec(shape, (lambda i: (0,) * nd))


_tc_mid = pl.pallas_call(
    _tc_mid_body,
    grid=(N // BN,),
    in_specs=[
        _row_block(NFEAT),
        _row_block(NFEAT),
        _split_block(16),
        _full_block((NFEAT, NHID)),
        _full_block((NFEAT, NHID)),
        _full_block((NHID,)),
        _full_block((NHID, NCLASS)),
        _full_block((NHID, NCLASS)),
        _full_block((NCLASS,)),
    ],
    out_specs=[_row_block(2 * NCLASS), _row_block(NCLASS)],
    out_shape=[
        jax.ShapeDtypeStruct((N, 2 * NCLASS), jnp.float32),
        jax.ShapeDtypeStruct((N, NCLASS), jnp.float32),
    ],
)


def _tc_out_body(self2_r, aggp2_r, degp_r, out_r):
  inv = _deg_inv(degp_r)
  logits = self2_r[...] + (
      aggp2_r[:, 0:NCLASS] + aggp2_r[:, NCLASS:2 * NCLASS]) * inv
  m = jnp.max(logits, axis=1, keepdims=True)
  ex = jnp.exp(logits - m)
  lse = jnp.log(jnp.sum(ex, axis=1, keepdims=True)) + m
  out_r[...] = logits - lse


_tc_out = pl.pallas_call(
    _tc_out_body,
    grid=(N // BN,),
    in_specs=[_row_block(NCLASS), _row_block(2 * NCLASS), _split_block(16)],
    out_specs=_row_block(NCLASS),
    out_shape=jax.ShapeDtypeStruct((N, NCLASS), jnp.float32),
)


def kernel(x, edge_index, W1_self, W1_neigh, b1, W2_self, W2_neigh, b2):
  ei = edge_index.astype(jnp.int32)
  src, dst = ei[0], ei[1]
  feat2 = x.reshape(2 * N, HF)
  src2 = (src * 2).reshape(NS, NCHA, CH)
  srcw2 = jnp.stack([src2, src2 + 1]).reshape(NC * NS, NCHA, CH)
  dstw_a = dst.reshape(NS, NCHA, CH)
  aggx, degp = _sc_layer1()(feat2, srcw2, dstw_a)
  degp = degp.reshape(NC, NPAD, 16)
  y2, self2 = _tc_mid(x, aggx, degp, W1_self, W1_neigh, b1,
                      W2_self, W2_neigh, b2)
  y2v = y2.reshape(2 * N, NCLASS)
  srcw_b = (src * 2).reshape(NW, NCHB, CH)
  dstw_b = dst.reshape(NW, NCHB, CH)
  (agg2,) = _sc_layer2()(y2v, srcw_b, dstw_b)
  return _tc_out(self2, agg2, degp)


# speedup vs baseline: 1.1902x; 1.0445x over previous
import jax
import jax.numpy as jnp
from jax import lax
from jax.experimental import pallas as pl
from jax.experimental.pallas import tpu as pltpu
from jax.experimental.pallas import tpu_sc as plsc

N = 10000
E = 320000
NFEAT = 128
NHID = 128
NCLASS = 64
HF = NFEAT // 2

NC = 2
NS = 16
NW = NC * NS
CH = 80
NCHA = E // NS // CH
NCHB = E // NW // CH
NPAD = 10240
RPT = NPAD // NS
ZR = 64

_HIGH = lax.Precision.DEFAULT
_SC_PARAMS = pltpu.CompilerParams(use_tc_tiling_on_sc=False)


def _memset_rows(ref, nrows, ncols, value):
  def row(r, _):
    def col(c, _):
      ref[r, pl.ds(c * 16, 16)] = jnp.full((16,), value, jnp.float32)
      return 0
    return lax.fori_loop(0, ncols // 16, col, 0)
  lax.fori_loop(0, nrows, row, 0)


def _agg_pipeline(nch, slots, lead, feat, src_v, dst_v, rows, acc,
                  gsem, ssem, deg_step=None, transform=None):
  def gather(j, slot):
    if transform is not None:
      transform(j)
    pltpu.async_copy(feat.at[src_v.at[j]], rows.at[slot], gsem)

  def wait_gather(slot):
    pltpu.make_async_copy(feat.at[src_v.at[0]], rows.at[slot], gsem).wait()

  def scatter(j, slot):
    pltpu.async_copy(rows.at[slot], acc.at[dst_v.at[j]], ssem, add=True)

  def drain_scatter(slot):
    pltpu.make_async_copy(rows.at[slot], acc.at[dst_v.at[0]], ssem).wait()

  for b in range(lead):
    gather(b, b)

  lag = slots - lead

  def round_(o, _):
    for b in range(slots):
      k = o * slots + b
      wait_gather(b)
      scatter(k, b)
      if deg_step is not None:
        deg_step(k)

      @pl.when(k >= lag)
      def _():
        drain_scatter(b)

      @pl.when(k + lead < nch)
      def _():
        gather(k + lead, (b + lead) % slots)
    return 0
  lax.fori_loop(0, nch // slots, round_, 0)
  for _ in range(lag):
    drain_scatter(0)


def _make_sc_layer1():
  mesh = plsc.VectorSubcoreMesh(core_axis_name="c", subcore_axis_name="s",
                                num_cores=NC, num_subcores=NS)
  out_type = [
      jax.ShapeDtypeStruct((NPAD, NFEAT), jnp.float32),
      jax.ShapeDtypeStruct((NC * NPAD, 16), jnp.float32),
  ]
  slots, lead = 5, 3
  scratch = [
      pltpu.VMEM((NCHA, CH), jnp.int32),
      pltpu.VMEM((NCHA, CH), jnp.int32),
      pltpu.VMEM((slots, CH, HF), jnp.float32),
      pltpu.VMEM((ZR, HF), jnp.float32),
      pltpu.VMEM((CH, 16), jnp.float32),
      pltpu.VMEM((ZR, 16), jnp.float32),
      pltpu.VMEM_SHARED((NPAD, HF), jnp.float32),
      pltpu.VMEM_SHARED((NPAD, 16), jnp.float32),
      pltpu.SemaphoreType.DMA,
      pltpu.SemaphoreType.DMA,
      pltpu.SemaphoreType.DMA,
  ]

  def body(feat2, srcw, dstw, agg_out, deg_out,
           src_v, dst_v, rows, zbuf, ones_v, zd, acc, dacc,
           gsem, ssem, dsem):
    cid = lax.axis_index("c")
    sid = lax.axis_index("s")

    _memset_rows(zbuf, ZR, HF, 0.0)
    _memset_rows(ones_v, CH, 16, 1.0)
    _memset_rows(zd, ZR, 16, 0.0)

    base = sid * RPT
    for k in range(RPT // ZR):
      pltpu.sync_copy(zbuf, acc.at[pl.ds(base + k * ZR, ZR)])
      pltpu.sync_copy(zd, dacc.at[pl.ds(base + k * ZR, ZR)])
    plsc.subcore_barrier()

    pltpu.sync_copy(srcw.at[sid], src_v)
    pltpu.sync_copy(dstw.at[sid], dst_v)

    def xform(j):
      for q in range(CH // 16):
        v = src_v[j, pl.ds(q * 16, 16)]
        src_v[j, pl.ds(q * 16, 16)] = v * 2 + cid

    def deg_step(k):
      @pl.when(k % 2 == cid)
      def _():
        pltpu.async_copy(ones_v, dacc.at[dst_v.at[k]], dsem, add=True)

      @pl.when((k % 2 == cid) & (k >= 2))
      def _():
        pltpu.make_async_copy(ones_v, dacc.at[dst_v.at[0]], dsem).wait()

    _agg_pipeline(NCHA, slots, lead, feat2, src_v, dst_v, rows, acc,
                  gsem, ssem, deg_step, xform)
    pltpu.make_async_copy(ones_v, dacc.at[dst_v.at[0]], dsem).wait()
    plsc.subcore_barrier()

    pltpu.sync_copy(acc.at[pl.ds(base, RPT)],
                    agg_out.at[pl.ds(base, RPT), pl.ds(cid * HF, HF)])
    obase = cid * NPAD + base
    pltpu.sync_copy(dacc.at[pl.ds(base, RPT)], deg_out.at[pl.ds(obase, RPT)])

  return pl.kernel(body, out_type=out_type, mesh=mesh, scratch_types=scratch,
                   compiler_params=_SC_PARAMS)


def _make_sc_layer2():
  mesh = plsc.VectorSubcoreMesh(core_axis_name="c", subcore_axis_name="s",
                                num_cores=NC, num_subcores=NS)
  out_type = [jax.ShapeDtypeStruct((NPAD, 2 * NCLASS), jnp.float32)]
  slots, lead = 5, 3
  scratch = [
      pltpu.VMEM((NCHB, CH), jnp.int32),
      pltpu.VMEM((NCHB, CH), jnp.int32),
      pltpu.VMEM((slots, CH, NCLASS), jnp.float32),
      pltpu.VMEM((ZR, NCLASS), jnp.float32),
      pltpu.VMEM_SHARED((NPAD, NCLASS), jnp.float32),
      pltpu.SemaphoreType.DMA,
      pltpu.SemaphoreType.DMA,
  ]

  def body(feat, srcw, dstw, agg_out, src_v, dst_v, rows, zbuf, acc,
           gsem, ssem):
    cid = lax.axis_index("c")
    sid = lax.axis_index("s")
    wid = sid * NC + cid

    _memset_rows(zbuf, ZR, NCLASS, 0.0)
    base = sid * RPT
    for k in range(RPT // ZR):
      pltpu.sync_copy(zbuf, acc.at[pl.ds(base + k * ZR, ZR)])
    plsc.subcore_barrier()

    pltpu.sync_copy(srcw.at[wid], src_v)
    pltpu.sync_copy(dstw.at[wid], dst_v)

    def xform(j):
      for q in range(CH // 16):
        v = src_v[j, pl.ds(q * 16, 16)]
        src_v[j, pl.ds(q * 16, 16)] = v * 2

    _agg_pipeline(NCHB, slots, lead, feat, src_v, dst_v, rows, acc,
                  gsem, ssem, None, xform)
    plsc.subcore_barrier()

    pltpu.sync_copy(acc.at[pl.ds(base, RPT)],
                    agg_out.at[pl.ds(base, RPT), pl.ds(cid * NCLASS, NCLASS)])

  return pl.kernel(body, out_type=out_type, mesh=mesh, scratch_types=scratch,
                   compiler_params=_SC_PARAMS)


_sc_cache = {}


def _sc_layer1():
  if "a" not in _sc_cache:
    _sc_cache["a"] = _make_sc_layer1()
  return _sc_cache["a"]


def _sc_layer2():
  if "b" not in _sc_cache:
    _sc_cache["b"] = _make_sc_layer2()
  return _sc_cache["b"]


BN = 2000


def _deg_inv(degp_ref):
  deg = degp_ref[0, :, 0:1] + degp_ref[1, :, 0:1]
  return 1.0 / jnp.maximum(deg, 1.0)


def _tc_mid_body(x_r, aggp_r, degp_r, w1s_r, w1n_r, b1_r, w2s_r, w2n_r,
                 b2_r, y2_r, self2_r):
  inv = _deg_inv(degp_r)
  agg = aggp_r[...] * inv
  x = x_r[...]
  h = (jnp.dot(x, w1s_r[...], preferred_element_type=jnp.float32,
               precision=_HIGH)
       + jnp.dot(agg, w1n_r[...], preferred_element_type=jnp.float32,
                 precision=_HIGH)
       + b1_r[...][None, :])
  h = jnp.maximum(h, 0.0)
  y2 = jnp.dot(h, w2n_r[...], preferred_element_type=jnp.float32,
               precision=_HIGH)
  y2_r[...] = jnp.concatenate(
      [y2, jnp.zeros((BN, NCLASS), jnp.float32)], axis=1)
  self2_r[...] = (jnp.dot(h, w2s_r[...], preferred_element_type=jnp.float32,
                          precision=_HIGH) + b2_r[...][None, :])


def _row_block(d):
  return pl.BlockSpec((BN, d), lambda i: (i, 0))


def _split_block(d):
  return pl.BlockSpec((2, BN, d), lambda i: (0, i, 0))


def _full_block(shape):
  nd = len(shape)
  return pl.BlockSpec(shape, (lambda i: (0,) * nd))


_tc_mid = pl.pallas_call(
    _tc_mid_body,
    grid=(N // BN,),
    in_specs=[
        _row_block(NFEAT),
        _row_block(NFEAT),
        _split_block(16),
        _full_block((NFEAT, NHID)),
        _full_block((NFEAT, NHID)),
        _full_block((NHID,)),
        _full_block((NHID, NCLASS)),
        _full_block((NHID, NCLASS)),
        _full_block((NCLASS,)),
    ],
    out_specs=[_row_block(2 * NCLASS), _row_block(NCLASS)],
    out_shape=[
        jax.ShapeDtypeStruct((N, 2 * NCLASS), jnp.float32),
        jax.ShapeDtypeStruct((N, NCLASS), jnp.float32),
    ],
)


def _tc_out_body(self2_r, aggp2_r, degp_r, out_r):
  inv = _deg_inv(degp_r)
  logits = self2_r[...] + (
      aggp2_r[:, 0:NCLASS] + aggp2_r[:, NCLASS:2 * NCLASS]) * inv
  m = jnp.max(logits, axis=1, keepdims=True)
  ex = jnp.exp(logits - m)
  lse = jnp.log(jnp.sum(ex, axis=1, keepdims=True)) + m
  out_r[...] = logits - lse


_tc_out = pl.pallas_call(
    _tc_out_body,
    grid=(N // BN,),
    in_specs=[_row_block(NCLASS), _row_block(2 * NCLASS), _split_block(16)],
    out_specs=_row_block(NCLASS),
    out_shape=jax.ShapeDtypeStruct((N, NCLASS), jnp.float32),
)


def kernel(x, edge_index, W1_self, W1_neigh, b1, W2_self, W2_neigh, b2):
  ei = edge_index.astype(jnp.int32)
  src, dst = ei[0], ei[1]
  feat2 = x.reshape(2 * N, HF)
  srcw_a = src.reshape(NS, NCHA, CH)
  dstw_a = dst.reshape(NS, NCHA, CH)
  aggx, degp = _sc_layer1()(feat2, srcw_a, dstw_a)
  degp = degp.reshape(NC, NPAD, 16)
  y2, self2 = _tc_mid(x, aggx, degp, W1_self, W1_neigh, b1,
                      W2_self, W2_neigh, b2)
  y2v = y2.reshape(2 * N, NCLASS)
  srcw_b = src.reshape(NW, NCHB, CH)
  dstw_b = dst.reshape(NW, NCHB, CH)
  (agg2,) = _sc_layer2()(y2v, srcw_b, dstw_b)
  return _tc_out(self2, agg2, degp)
